# baseline (speedup 1.0000x reference)
import functools

import jax
import jax.numpy as jnp
from jax import lax
from jax.experimental import pallas as pl
from jax.experimental.pallas import tpu as pltpu
from jax.experimental.pallas import tpu_sc as plsc

_N_SLICES = 2


def _make_sc_gather(n_rows, d, dtype):
    info = plsc.get_sparse_core_info()
    nw = info.num_cores * info.num_subcores
    assert n_rows % nw == 0
    b_per_w = n_rows // nw
    ch = b_per_w
    while ch * d * 4 > 224 * 1024 or b_per_w % ch:
        ch -= 1
    nchunk = b_per_w // ch
    assert ch % 8 == 0 and b_per_w % 8 == 0

    mesh = plsc.VectorSubcoreMesh(core_axis_name="c", subcore_axis_name="s")

    @functools.partial(
        pl.kernel,
        mesh=mesh,
        out_type=jax.ShapeDtypeStruct((n_rows, d), dtype),
        scratch_types=[
            pltpu.VMEM((b_per_w,), jnp.int32),
            pltpu.VMEM((ch, d), dtype),
            pltpu.VMEM((ch, d), dtype),
            pltpu.SemaphoreType.DMA,
            pltpu.SemaphoreType.DMA,
            pltpu.SemaphoreType.DMA,
            pltpu.SemaphoreType.DMA,
        ],
    )
    def gather_k(table_hbm, idx_hbm, out_hbm, idx_v,
                 buf0, buf1, gsem0, gsem1, ssem0, ssem1):
        wid = lax.axis_index("s") * info.num_cores + lax.axis_index("c")
        base = wid * b_per_w
        pltpu.sync_copy(idx_hbm.at[pl.ds(base, b_per_w)], idx_v)
        bufs = (buf0, buf1)
        gsems = (gsem0, gsem1)
        ssems = (ssem0, ssem1)
        gathers = [
            pltpu.async_copy(
                table_hbm.at[idx_v.at[pl.ds(0, ch)]], buf0, gsem0)
        ]
        stores = []
        for c in range(nchunk):
            if c + 1 < nchunk:
                if c >= 1:
                    stores[c - 1].wait()
                gathers.append(pltpu.async_copy(
                    table_hbm.at[idx_v.at[pl.ds((c + 1) * ch, ch)]],
                    bufs[(c + 1) % 2], gsems[(c + 1) % 2]))
            gathers[c].wait()
            stores.append(pltpu.async_copy(
                bufs[c % 2], out_hbm.at[pl.ds(base + c * ch, ch)],
                ssems[c % 2]))
        for st in stores[-2:]:
            st.wait()

    return gather_k


def _tc_body(is_first, is_last,
             g_ref, afe_ref, wca_ref, wcb_ref, bc_ref, sel_ref,
             prev_ref, out_ref):
    i = pl.program_id(0)
    n = pl.num_programs(0)
    x = g_ref[0]
    afet = afe_ref[0]
    embt = lax.dot_general(afet, x, (((1,), (1,)), ((), ())),
                           preferred_element_type=jnp.float32)
    dp = embt.shape[0] // 2
    sst = jnp.dot(sel_ref[...], embt * embt,
                  preferred_element_type=jnp.float32)
    rt = 1.0 / jnp.maximum(jnp.sqrt(sst), 1e-12)
    u0 = jnp.dot(wca_ref[0], embt[:dp], preferred_element_type=jnp.float32)
    u1 = jnp.dot(wcb_ref[0], embt[dp:], preferred_element_type=jnp.float32)
    contrib = u0 * rt[0:1, :] + u1 * rt[1:2, :]

    @pl.when(i == 0)
    def _():
        if is_first:
            out_ref[...] = contrib
        else:
            out_ref[...] = prev_ref[...] + contrib

    @pl.when(i > 0)
    def _():
        out_ref[...] = out_ref[...] + contrib

    if is_last:
        @pl.when(i == n - 1)
        def _():
            logits = out_ref[...] + bc_ref[...]
            m = jnp.max(logits, axis=0, keepdims=True)
            e = jnp.exp(logits - m)
            out_ref[...] = e / jnp.sum(e, axis=0, keepdims=True)


def _tc_slice(g, afet_all, wc3t, bct, selt, prev, offset, n_win,
              is_first, is_last):
    _, b, d = g.shape
    dpp = afet_all.shape[1]
    nc = wc3t.shape[1]
    dp = wc3t.shape[2]
    if offset == 0:
        afe_ix = lambda i: (jnp.minimum(i, 1), 0, 0)
        wca_ix = lambda i: (jnp.where(i == 0, 0, 1 + i), 0, 0)
        wcb_ix = lambda i: (jnp.where(i == 0, 1, 26 + i), 0, 0)
    else:
        afe_ix = lambda i: (1, 0, 0)
        wca_ix = lambda i: (1 + offset + i, 0, 0)
        wcb_ix = lambda i: (26 + offset + i, 0, 0)
    return pl.pallas_call(
        functools.partial(_tc_body, is_first, is_last),
        grid=(n_win,),
        in_specs=[
            pl.BlockSpec((1, b, d), lambda i, _o=offset: (i + _o, 0, 0)),
            pl.BlockSpec((1, dpp, d), afe_ix),
            pl.BlockSpec((1, nc, dp), wca_ix),
            pl.BlockSpec((1, nc, dp), wcb_ix),
            pl.BlockSpec((nc, b), lambda i: (0, 0)),
            pl.BlockSpec((2, dpp), lambda i: (0, 0)),
            pl.BlockSpec((nc, b), lambda i: (0, 0)),
        ],
        out_specs=pl.BlockSpec((nc, b), lambda i: (0, 0)),
        out_shape=jax.ShapeDtypeStruct((nc, b), jnp.float32),
        compiler_params=pltpu.CompilerParams(
            dimension_semantics=("arbitrary",)),
    )(g, afet_all, wc3t, wc3t, bct, selt, prev)


def kernel(features, AFE_a, AFE_r, Wc, bc, c_ids, nei_ids):
    n_nodes, d = features.shape
    b = c_ids.shape[0]
    s = nei_ids.shape[1]
    n_afe_a = AFE_a.shape[0]
    n_afe_r = AFE_r.shape[0]
    dp = AFE_a.shape[2]
    nc = Wc.shape[1]
    n_steps = 1 + s

    idx_all = jnp.concatenate(
        [c_ids.astype(jnp.int32), nei_ids.T.reshape(-1).astype(jnp.int32)])

    afet_all = jnp.stack(
        [jnp.concatenate([AFE_a[k].T for k in range(n_afe_a)], axis=0),
         jnp.concatenate([AFE_r[k].T for k in range(n_afe_r)], axis=0)])

    wc3t = Wc.reshape(n_afe_a + n_afe_r * s, dp, nc).transpose(0, 2, 1)
    bct = jnp.broadcast_to(bc.reshape(nc, 1), (nc, b))
    selt = (jnp.arange(2)[:, None]
            == jnp.arange(n_afe_r * dp)[None, :] // dp).astype(jnp.float32)

    g = _make_sc_gather(n_steps * b, d, features.dtype)(
        features, idx_all).reshape(n_steps, b, d)

    half = n_steps // 2
    logits = jnp.zeros((nc, b), jnp.float32)
    logits = _tc_slice(g, afet_all, wc3t, bct, selt, logits,
                       offset=0, n_win=half, is_first=True, is_last=False)
    logits = _tc_slice(g, afet_all, wc3t, bct, selt, logits,
                       offset=half, n_win=n_steps - half,
                       is_first=False, is_last=True)

    return logits.T


# speedup vs baseline: 1.6353x; 1.6353x over previous
import functools

import jax
import jax.numpy as jnp
from jax import lax
from jax.experimental import pallas as pl
from jax.experimental.pallas import tpu as pltpu
from jax.experimental.pallas import tpu_sc as plsc

_N_SLICES = 2


def _make_sc_gather(n_rows, d, dtype):
    info = plsc.get_sparse_core_info()
    nw = info.num_cores * info.num_subcores
    assert n_rows % nw == 0
    b_per_w = n_rows // nw
    ch = b_per_w
    while ch * d * 4 > 224 * 1024 or b_per_w % ch:
        ch -= 1
    nchunk = b_per_w // ch
    assert ch % 8 == 0 and b_per_w % 8 == 0

    mesh = plsc.VectorSubcoreMesh(core_axis_name="c", subcore_axis_name="s")

    @functools.partial(
        pl.kernel,
        mesh=mesh,
        out_type=jax.ShapeDtypeStruct((n_rows, d), dtype),
        scratch_types=[
            pltpu.VMEM((b_per_w,), jnp.int32),
            pltpu.VMEM((ch, d), dtype),
            pltpu.VMEM((ch, d), dtype),
            pltpu.SemaphoreType.DMA,
            pltpu.SemaphoreType.DMA,
            pltpu.SemaphoreType.DMA,
            pltpu.SemaphoreType.DMA,
        ],
    )
    def gather_k(table_hbm, idx_hbm, out_hbm, idx_v,
                 buf0, buf1, gsem0, gsem1, ssem0, ssem1):
        wid = lax.axis_index("s") * info.num_cores + lax.axis_index("c")
        base = wid * b_per_w
        pltpu.sync_copy(idx_hbm.at[pl.ds(base, b_per_w)], idx_v)
        bufs = (buf0, buf1)
        gsems = (gsem0, gsem1)
        ssems = (ssem0, ssem1)
        gathers = [
            pltpu.async_copy(
                table_hbm.at[idx_v.at[pl.ds(0, ch)]], buf0, gsem0)
        ]
        stores = []
        for c in range(nchunk):
            if c + 1 < nchunk:
                if c >= 1:
                    stores[c - 1].wait()
                gathers.append(pltpu.async_copy(
                    table_hbm.at[idx_v.at[pl.ds((c + 1) * ch, ch)]],
                    bufs[(c + 1) % 2], gsems[(c + 1) % 2]))
            gathers[c].wait()
            stores.append(pltpu.async_copy(
                bufs[c % 2], out_hbm.at[pl.ds(base + c * ch, ch)],
                ssems[c % 2]))
        for st in stores[-2:]:
            st.wait()

    return gather_k


def _tc_body(is_first, is_last,
             g_ref, afe_ref, wca_ref, wcb_ref, bc_ref, sel_ref,
             prev_ref, out_ref):
    i = pl.program_id(0)
    n = pl.num_programs(0)
    x = g_ref[0]
    afet = afe_ref[0]
    embt = lax.dot_general(afet, x, (((1,), (1,)), ((), ())),
                           preferred_element_type=jnp.float32)
    dp = embt.shape[0] // 2
    sst = jnp.dot(sel_ref[...], embt * embt,
                  preferred_element_type=jnp.float32)
    rt = 1.0 / jnp.maximum(jnp.sqrt(sst), 1e-12)
    u0 = jnp.dot(wca_ref[0], embt[:dp], preferred_element_type=jnp.float32)
    u1 = jnp.dot(wcb_ref[0], embt[dp:], preferred_element_type=jnp.float32)
    contrib = u0 * rt[0:1, :] + u1 * rt[1:2, :]

    @pl.when(i == 0)
    def _():
        if is_first:
            out_ref[...] = contrib
        else:
            out_ref[...] = prev_ref[...] + contrib

    @pl.when(i > 0)
    def _():
        out_ref[...] = out_ref[...] + contrib

    if is_last:
        @pl.when(i == n - 1)
        def _():
            logits = out_ref[...] + bc_ref[...]
            m = jnp.max(logits, axis=0, keepdims=True)
            e = jnp.exp(logits - m)
            out_ref[...] = e / jnp.sum(e, axis=0, keepdims=True)


def _tc_slice(g, afet_all, wc3t, bct, selt, prev, offset, is_first, is_last):
    n_win, b, d = g.shape
    dpp = afet_all.shape[1]
    nc = wc3t.shape[1]
    dp = wc3t.shape[2]
    if offset == 0:
        afe_ix = lambda i: (jnp.minimum(i, 1), 0, 0)
        wca_ix = lambda i: (jnp.where(i == 0, 0, 1 + i), 0, 0)
        wcb_ix = lambda i: (jnp.where(i == 0, 1, 26 + i), 0, 0)
    else:
        afe_ix = lambda i: (1, 0, 0)
        wca_ix = lambda i: (1 + offset + i, 0, 0)
        wcb_ix = lambda i: (26 + offset + i, 0, 0)
    return pl.pallas_call(
        functools.partial(_tc_body, is_first, is_last),
        grid=(n_win,),
        in_specs=[
            pl.BlockSpec((1, b, d), lambda i: (i, 0, 0)),
            pl.BlockSpec((1, dpp, d), afe_ix),
            pl.BlockSpec((1, nc, dp), wca_ix),
            pl.BlockSpec((1, nc, dp), wcb_ix),
            pl.BlockSpec((nc, b), lambda i: (0, 0)),
            pl.BlockSpec((2, dpp), lambda i: (0, 0)),
            pl.BlockSpec((nc, b), lambda i: (0, 0)),
        ],
        out_specs=pl.BlockSpec((nc, b), lambda i: (0, 0)),
        out_shape=jax.ShapeDtypeStruct((nc, b), jnp.float32),
        compiler_params=pltpu.CompilerParams(
            dimension_semantics=("arbitrary",)),
    )(g, afet_all, wc3t, wc3t, bct, selt, prev)


def kernel(features, AFE_a, AFE_r, Wc, bc, c_ids, nei_ids):
    n_nodes, d = features.shape
    b = c_ids.shape[0]
    s = nei_ids.shape[1]
    n_afe_a = AFE_a.shape[0]
    n_afe_r = AFE_r.shape[0]
    dp = AFE_a.shape[2]
    nc = Wc.shape[1]
    n_steps = 1 + s

    idx_all = jnp.concatenate(
        [c_ids.astype(jnp.int32), nei_ids.T.reshape(-1).astype(jnp.int32)])

    afet_all = jnp.stack(
        [jnp.concatenate([AFE_a[k].T for k in range(n_afe_a)], axis=0),
         jnp.concatenate([AFE_r[k].T for k in range(n_afe_r)], axis=0)])

    wc3t = Wc.reshape(n_afe_a + n_afe_r * s, dp, nc).transpose(0, 2, 1)
    bct = jnp.broadcast_to(bc.reshape(nc, 1), (nc, b))
    selt = (jnp.arange(2)[:, None]
            == jnp.arange(n_afe_r * dp)[None, :] // dp).astype(jnp.float32)

    assert n_steps % _N_SLICES == 0
    sz = n_steps // _N_SLICES
    offsets = [k * sz for k in range(_N_SLICES)]
    gather = _make_sc_gather(sz * b, d, features.dtype)
    g_slices = [
        gather(features, idx_all[o * b:(o + sz) * b]).reshape(sz, b, d)
        for o in offsets
    ]

    logits = jnp.zeros((nc, b), jnp.float32)
    for k in range(_N_SLICES):
        logits = _tc_slice(
            g_slices[k], afet_all, wc3t, bct, selt, logits,
            offset=offsets[k], is_first=(k == 0),
            is_last=(k == _N_SLICES - 1))

    return logits.T
